# baseline (speedup 1.0000x reference)
import functools

import jax
import jax.numpy as jnp
from jax import lax
from jax.experimental import pallas as pl
from jax.experimental.pallas import tpu as pltpu
from jax.experimental.pallas import tpu_sc as plsc

VOCAB = 32000
D_MODEL = 4096
NUM_DEV = 8
D_SLICE = D_MODEL // NUM_DEV

_NC = 2
_NS = 16
_NW = _NC * _NS
_L = 16

TOKENS = 2 * 8192
ROWS = NUM_DEV * TOKENS
PT = ROWS // _NW
R = 64
NCH = PT // R


def _body(table, idx, out, xv, idx8, buf, sem):
    c = lax.axis_index("c")
    s = lax.axis_index("s")
    wid = s * _NC + c
    base = wid * PT
    d = base // TOKENS
    tok = base - d * TOKENS

    pltpu.sync_copy(idx.at[pl.ds(tok, PT)], xv)

    def compute_idx(j, carry):
        v = xv[pl.ds(j * _L, _L)]
        idx8[pl.ds(j * _L, _L)] = v * NUM_DEV + d
        return carry

    lax.fori_loop(0, PT // _L, compute_idx, 0, unroll=4)

    def chunk(i, carry):
        pltpu.async_copy(
            table.at[idx8.at[pl.ds(i * R, R)]], buf, sem
        ).wait()
        pltpu.sync_copy(buf, out.at[pl.ds(base + i * R, R)])
        return carry

    lax.fori_loop(0, NCH, chunk, 0)


@functools.partial(
    pl.kernel,
    out_type=jax.ShapeDtypeStruct((ROWS, D_SLICE), jnp.float32),
    mesh=plsc.VectorSubcoreMesh(core_axis_name="c", subcore_axis_name="s"),
    scratch_types=[
        pltpu.VMEM((PT,), jnp.int32),
        pltpu.VMEM((PT,), jnp.int32),
        pltpu.VMEM((R, D_SLICE), jnp.float32),
        pltpu.SemaphoreType.DMA,
    ],
)
def _emb_gather(table, idx, out, xv, idx8, buf, sem):
    _body(table, idx, out, xv, idx8, buf, sem)


def kernel(x, emb_weight):
    b, sq = x.shape
    table = emb_weight.reshape(VOCAB * NUM_DEV, D_SLICE)
    out = _emb_gather(table, x.reshape(-1))
    return out.reshape(NUM_DEV, b, sq, D_SLICE)


# speedup vs baseline: 4.0575x; 4.0575x over previous
import functools

import jax
import jax.numpy as jnp
from jax import lax
from jax.experimental import pallas as pl
from jax.experimental.pallas import tpu as pltpu
from jax.experimental.pallas import tpu_sc as plsc

VOCAB = 32000
D_MODEL = 4096
NUM_DEV = 8
D_SLICE = D_MODEL // NUM_DEV

_NC = 2
_NS = 16
_NW = _NC * _NS

TOKENS = 2 * 8192
ROWS = NUM_DEV * TOKENS
PT = ROWS // _NW
R = 64
NCH = PT // R


def _body(table, idx, out, xv, buf0, buf1, sem0, sem1):
    c = lax.axis_index("c")
    s = lax.axis_index("s")
    wid = s * _NC + c
    base = wid * PT
    d = base // TOKENS
    tok = base - d * TOKENS
    col = d * D_SLICE

    pltpu.sync_copy(idx.at[pl.ds(tok, PT)], xv)

    def gather(i, buf, sem):
        return pltpu.make_async_copy(
            table.at[xv.at[pl.ds(i * R, R)], pl.ds(col, D_SLICE)], buf, sem
        )

    def write(i, buf):
        pltpu.sync_copy(buf, out.at[pl.ds(base + i * R, R)])

    gather(0, buf0, sem0).start()

    def pair(i, carry):
        c0 = i * 2
        gather(c0 + 1, buf1, sem1).start()
        gather(c0, buf0, sem0).wait()
        write(c0, buf0)

        @pl.when(c0 + 2 < NCH)
        def _():
            gather(c0 + 2, buf0, sem0).start()

        gather(c0 + 1, buf1, sem1).wait()
        write(c0 + 1, buf1)
        return carry

    lax.fori_loop(0, NCH // 2, pair, 0)


@functools.partial(
    pl.kernel,
    out_type=jax.ShapeDtypeStruct((ROWS, D_SLICE), jnp.float32),
    mesh=plsc.VectorSubcoreMesh(core_axis_name="c", subcore_axis_name="s"),
    scratch_types=[
        pltpu.VMEM((PT,), jnp.int32),
        pltpu.VMEM((R, D_SLICE), jnp.float32),
        pltpu.VMEM((R, D_SLICE), jnp.float32),
        pltpu.SemaphoreType.DMA,
        pltpu.SemaphoreType.DMA,
    ],
)
def _emb_gather(table, idx, out, xv, buf0, buf1, sem0, sem1):
    _body(table, idx, out, xv, buf0, buf1, sem0, sem1)


def kernel(x, emb_weight):
    b, sq = x.shape
    out = _emb_gather(emb_weight, x.reshape(-1))
    return out.reshape(NUM_DEV, b, sq, D_SLICE)
